# Initial kernel scaffold; baseline (speedup 1.0000x reference)
import jax
import jax.numpy as jnp
from jax.experimental import pallas as pl


def kernel(x, Wr, br, W1, b1, W2, b2):
    raise NotImplementedError("write your pallas kernel here")


# speedup vs baseline: 8.3096x; 8.3096x over previous
import functools

import jax
import jax.numpy as jnp
from jax.experimental import pallas as pl
from jax.experimental.pallas import tpu as pltpu

N = 4096
D = 768
E = 16
FF = 1024

BN = 1024


def _moe_body(x_ref, wr_ref, br_ref, w1_ref, b1_ref, w2_ref, b2_ref,
              out_ref, wscr):
    e = pl.program_id(1)

    @pl.when(e == 0)
    def _router():
        x = x_ref[...]
        logits = jnp.dot(x, wr_ref[...], preferred_element_type=jnp.float32)
        logits = logits + br_ref[...]
        iota = jax.lax.broadcasted_iota(jnp.int32, (BN, E), 1)
        m1 = jnp.max(logits, axis=1, keepdims=True)
        idx1 = jnp.min(jnp.where(logits == m1, iota, E), axis=1, keepdims=True)
        l2 = jnp.where(iota == idx1, -jnp.inf, logits)
        m2 = jnp.max(l2, axis=1, keepdims=True)
        idx2 = jnp.min(jnp.where(l2 == m2, iota, E), axis=1, keepdims=True)
        b = jnp.exp(m2 - m1)
        w1 = 1.0 / (1.0 + b)
        w2 = b / (1.0 + b)
        w = jnp.where(iota == idx1, w1, jnp.where(iota == idx2, w2, 0.0))
        wscr[...] = w
        out_ref[...] = x

    x = x_ref[...]
    h = jnp.dot(x, w1_ref[0], preferred_element_type=jnp.float32) + b1_ref[0]
    h = jnp.maximum(h, 0.0)
    y = jnp.dot(h, w2_ref[0], preferred_element_type=jnp.float32) + b2_ref[0]
    lane = jax.lax.broadcasted_iota(jnp.int32, (BN, E), 1)
    w_e = jnp.sum(jnp.where(lane == e, wscr[...], 0.0), axis=1, keepdims=True)
    out_ref[...] += w_e * y


def kernel(x, Wr, br, W1, b1, W2, b2):
    br2 = br.reshape(1, E)
    b1r = b1.reshape(E, 1, FF)
    b2r = b2.reshape(E, 1, D)
    grid = (N // BN, E)
    out = pl.pallas_call(
        _moe_body,
        grid=grid,
        in_specs=[
            pl.BlockSpec((BN, D), lambda i, e: (i, 0)),
            pl.BlockSpec((D, E), lambda i, e: (0, 0)),
            pl.BlockSpec((1, E), lambda i, e: (0, 0)),
            pl.BlockSpec((1, D, FF), lambda i, e: (e, 0, 0)),
            pl.BlockSpec((1, 1, FF), lambda i, e: (e, 0, 0)),
            pl.BlockSpec((1, FF, D), lambda i, e: (e, 0, 0)),
            pl.BlockSpec((1, 1, D), lambda i, e: (e, 0, 0)),
        ],
        out_specs=pl.BlockSpec((BN, D), lambda i, e: (i, 0)),
        out_shape=jax.ShapeDtypeStruct((N, D), jnp.float32),
        scratch_shapes=[pltpu.VMEM((BN, E), jnp.float32)],
    )(x, Wr, br2, W1, b1r, W2, b2r)
    return out
